# baseline (speedup 1.0000x reference)
import functools

import jax
import jax.numpy as jnp
from jax.experimental import pallas as pl
from jax.experimental.pallas import tpu as pltpu
from jax.experimental.pallas import tpu_sc as plsc

K = 32


def _enc_topk_kernel(x_ref, w_ref, b_ref, z_ref, thr_ref, z_s, *, nd, dt):
    j = pl.program_id(1)
    x = x_ref[...]
    w = w_ref[...]
    z = jax.lax.dot_general(x, w, (((1,), (1,)), ((), ())),
                            preferred_element_type=jnp.float32)
    z = z + b_ref[...]
    z_s[j] = z
    z_ref[...] = z

    @pl.when(j == nd - 1)
    def _():
        zv = z_s[...]
        tb = zv.shape[1]

        qpt = -(-K // nd)
        cw = dt // qpt
        cms = []
        for jj in range(nd):
            zj = z_s[jj]
            for q in range(qpt):
                cms.append(jnp.max(zj[:, q * cw:(q + 1) * cw], axis=1,
                                   keepdims=True))
        lo0 = cms[0]
        hi0 = cms[0]
        for c in cms[1:]:
            lo0 = jnp.minimum(lo0, c)
            hi0 = jnp.maximum(hi0, c)

        kf = jnp.float32(K)

        def cond(c):
            i, lo, hi, cl = c
            return jnp.logical_and(i < 40, jnp.any(cl != kf))

        def body(c):
            i, lo, hi, cl = c
            mid = 0.5 * (lo + hi)
            m = (zv >= mid[None, :, :]).astype(jnp.float32)
            c1 = jnp.sum(m, axis=2)
            cnt = jnp.sum(c1, axis=0)[:, None]
            ge = cnt >= kf
            return (i + 1,
                    jnp.where(ge, mid, lo),
                    jnp.where(ge, hi, mid),
                    jnp.where(ge, cnt, cl))

        _, thr, _, _ = jax.lax.while_loop(
            cond, body,
            (jnp.int32(0), lo0, hi0, jnp.full((tb, 1), kf + 1.0, jnp.float32)))

        thr_ref[...] = jnp.broadcast_to(thr.reshape(1, tb, 1),
                                        thr_ref.shape)


def _dec_kernel(z_ref, thr_ref, w_ref, b_ref, out_ref, acc, *, nd):
    j = pl.program_id(1)

    @pl.when(j == 0)
    def _():
        acc[...] = jnp.zeros_like(acc)

    z = z_ref[...]
    thr = thr_ref[0, :, 0:1]
    zsp = jnp.where(z >= thr, z, 0.0)
    acc[...] += jax.lax.dot_general(zsp, w_ref[...],
                                    (((1,), (1,)), ((), ())),
                                    preferred_element_type=jnp.float32)

    @pl.when(j == nd - 1)
    def _():
        out_ref[...] = acc[...] + b_ref[...]


def _sc_mask_kernel(z_hbm, thr_hbm, out_hbm, row_v, thr_v, *, rows_per_w, rb):
    c = jax.lax.axis_index("c")
    s = jax.lax.axis_index("s")
    nc = jax.lax.axis_size("c")
    wid = s * nc + c
    base = wid * rows_per_w
    pltpu.sync_copy(thr_hbm.at[pl.ds(base * 16, rows_per_w * 16)], thr_v)

    def batch_body(b, carry):
        rbase = base + b * rb
        pltpu.sync_copy(z_hbm.at[pl.ds(rbase, rb)], row_v)
        for rr in range(rb):
            rl = b * rb + rr
            thrs = thr_v[pl.ds(rl * 16, 16)]

            def inner(v, _):
                for u in range(8):
                    idx = pl.ds((v * 8 + u) * 16, 16)
                    xv = row_v[rr, idx]
                    row_v[rr, idx] = jnp.where(xv >= thrs, xv, 0.0)
                return 0

            jax.lax.fori_loop(0, 64, inner, 0)
        pltpu.sync_copy(row_v, out_hbm.at[pl.ds(rbase, rb)])
        return carry

    jax.lax.fori_loop(0, rows_per_w // rb, batch_body, 0)


def kernel(x, W_enc, b_enc, W_dec, b_dec):
    n_chunks = 2
    nt_all = x.shape[0]
    cs = nt_all // n_chunks
    outs = [_chunk(x[c * cs:(c + 1) * cs], W_enc, b_enc, W_dec, b_dec)
            for c in range(n_chunks)]
    x_hat = jnp.concatenate([o[0] for o in outs], axis=0)
    z_sparse = jnp.concatenate([o[1] for o in outs], axis=0)
    return (x_hat, z_sparse)


def _chunk(x, W_enc, b_enc, W_dec, b_dec):
    n_tok, d_in = x.shape
    d_dict = W_enc.shape[0]
    tb = min(256, n_tok)
    dt = 1024
    nt = n_tok // tb
    nd = d_dict // dt
    b_enc2 = b_enc.reshape(1, d_dict)
    b_dec2 = b_dec.reshape(1, d_in)

    z, thr = pl.pallas_call(
        functools.partial(_enc_topk_kernel, nd=nd, dt=dt),
        grid=(nt, nd),
        in_specs=[
            pl.BlockSpec((tb, d_in), lambda i, j: (i, 0)),
            pl.BlockSpec((dt, d_in), lambda i, j: (j, 0)),
            pl.BlockSpec((1, dt), lambda i, j: (0, j)),
        ],
        out_specs=[
            pl.BlockSpec((tb, dt), lambda i, j: (i, j)),
            pl.BlockSpec((1, tb, 16), lambda i, j: (i, 0, 0)),
        ],
        out_shape=[
            jax.ShapeDtypeStruct((n_tok, d_dict), jnp.float32),
            jax.ShapeDtypeStruct((nt, tb, 16), jnp.float32),
        ],
        scratch_shapes=[pltpu.VMEM((nd, tb, dt), jnp.float32)],
    )(x, W_enc, b_enc2)

    thr_flat = thr.reshape(n_tok * 16)

    info = plsc.get_sparse_core_info()
    nw = info.num_cores * info.num_subcores
    rows_per_w = n_tok // nw
    rb = min(8, rows_per_w)

    mesh = plsc.VectorSubcoreMesh(core_axis_name="c", subcore_axis_name="s")
    sc_mask = functools.partial(
        pl.kernel,
        mesh=mesh,
        out_type=jax.ShapeDtypeStruct((n_tok, d_dict), jnp.float32),
        scratch_types=[
            pltpu.VMEM((rb, d_dict), jnp.float32),
            pltpu.VMEM((rows_per_w * 16,), jnp.float32),
        ],
    )(functools.partial(_sc_mask_kernel, rows_per_w=rows_per_w, rb=rb))
    z_sparse = sc_mask(z, thr_flat)

    x_hat = pl.pallas_call(
        functools.partial(_dec_kernel, nd=nd),
        grid=(nt, nd),
        in_specs=[
            pl.BlockSpec((tb, dt), lambda i, j: (i, j)),
            pl.BlockSpec((1, tb, 16), lambda i, j: (i, 0, 0)),
            pl.BlockSpec((d_in, dt), lambda i, j: (0, j)),
            pl.BlockSpec((1, d_in), lambda i, j: (0, 0)),
        ],
        out_specs=pl.BlockSpec((tb, d_in), lambda i, j: (i, 0)),
        out_shape=jax.ShapeDtypeStruct((n_tok, d_in), jnp.float32),
        scratch_shapes=[pltpu.VMEM((tb, d_in), jnp.float32)],
    )(z, thr, W_dec, b_dec2)

    return (x_hat, z_sparse)


# speedup vs baseline: 1.2744x; 1.2744x over previous
import functools

import jax
import jax.numpy as jnp
from jax.experimental import pallas as pl
from jax.experimental.pallas import tpu as pltpu

K = 32
F_ITERS = 2


def _count_ge(zv, mid):
    m = (zv >= mid[None, :, :]).astype(jnp.float32)
    c1 = jnp.sum(m, axis=2)
    return jnp.sum(c1, axis=0)[:, None]


def _enc_topk_kernel(x_ref, w_ref, b_ref, out_ref,
                     z_s, bm_lo, bm_hi, st_lo, st_hi, st_cl,
                     *, nt, nd, dt, tb):
    i = pl.program_id(0)
    j = pl.program_id(1)
    cur = jax.lax.rem(i, 2)
    prev = jax.lax.rem(i + 1, 2)
    kf = jnp.float32(K)
    qpt = -(-K // nd)
    cw = dt // qpt

    @pl.when(i < nt)
    def _encode():
        z = jax.lax.dot_general(x_ref[...], w_ref[...],
                                (((1,), (1,)), ((), ())),
                                preferred_element_type=jnp.float32)
        z = z + b_ref[...]
        z_s[cur, j] = z
        cmn = jnp.max(z[:, 0:cw], axis=1, keepdims=True)
        cmx = cmn
        for q in range(1, qpt):
            m_q = jnp.max(z[:, q * cw:(q + 1) * cw], axis=1, keepdims=True)
            cmn = jnp.minimum(cmn, m_q)
            cmx = jnp.maximum(cmx, m_q)
        bm_lo[cur] = jnp.where(j == 0, cmn, jnp.minimum(bm_lo[cur], cmn))
        bm_hi[cur] = jnp.where(j == 0, cmx, jnp.maximum(bm_hi[cur], cmx))

    @pl.when(i >= 1)
    def _bisect():
        zv = z_s[prev]

        @pl.when(jnp.any(st_cl[...] != kf))
        def _steps():
            lo = st_lo[...]
            hi = st_hi[...]
            cl = st_cl[...]
            for _ in range(F_ITERS):
                mid = 0.5 * (lo + hi)
                cnt = _count_ge(zv, mid)
                ge = cnt >= kf
                lo = jnp.where(ge, mid, lo)
                hi = jnp.where(ge, hi, mid)
                cl = jnp.where(ge, cnt, cl)
            st_lo[...] = lo
            st_hi[...] = hi
            st_cl[...] = cl

        @pl.when(j == nd - 1)
        def _finish():
            def cond(c):
                it, lo, hi, cl = c
                return jnp.logical_and(it < 40, jnp.any(cl != kf))

            def body(c):
                it, lo, hi, cl = c
                mid = 0.5 * (lo + hi)
                cnt = _count_ge(zv, mid)
                ge = cnt >= kf
                return (it + 1,
                        jnp.where(ge, mid, lo),
                        jnp.where(ge, hi, mid),
                        jnp.where(ge, cnt, cl))

            _, thr, _, _ = jax.lax.while_loop(
                cond, body,
                (jnp.int32(0), st_lo[...], st_hi[...], st_cl[...]))

            for jj in range(nd):
                zj = z_s[prev, jj]
                out_ref[:, jj * dt:(jj + 1) * dt] = \
                    jnp.where(zj >= thr, zj, 0.0)

    @pl.when(jnp.logical_and(i < nt, j == nd - 1))
    def _seed():
        st_lo[...] = bm_lo[cur]
        st_hi[...] = bm_hi[cur]
        st_cl[...] = jnp.full((tb, 1), kf + 1.0, jnp.float32)


def _dec_kernel(zs_ref, w_ref, b_ref, out_ref, acc, *, nd):
    j = pl.program_id(1)

    @pl.when(j == 0)
    def _():
        acc[...] = jnp.zeros_like(acc)

    acc[...] += jax.lax.dot_general(zs_ref[...], w_ref[...],
                                    (((1,), (1,)), ((), ())),
                                    preferred_element_type=jnp.float32)

    @pl.when(j == nd - 1)
    def _():
        out_ref[...] = acc[...] + b_ref[...]


def kernel(x, W_enc, b_enc, W_dec, b_dec):
    n_tok, d_in = x.shape
    d_dict = W_enc.shape[0]
    tb = min(256, n_tok)
    dt = 1024
    nt = n_tok // tb
    nd = d_dict // dt
    b_enc2 = b_enc.reshape(1, d_dict)
    b_dec2 = b_dec.reshape(1, d_in)

    z_sparse = pl.pallas_call(
        functools.partial(_enc_topk_kernel, nt=nt, nd=nd, dt=dt, tb=tb),
        grid=(nt + 1, nd),
        in_specs=[
            pl.BlockSpec((tb, d_in),
                         lambda i, j: (jnp.minimum(i, nt - 1), 0)),
            pl.BlockSpec((dt, d_in), lambda i, j: (j, 0)),
            pl.BlockSpec((1, dt), lambda i, j: (0, j)),
        ],
        out_specs=pl.BlockSpec((tb, d_dict),
                               lambda i, j: (jnp.maximum(i - 1, 0), 0)),
        out_shape=jax.ShapeDtypeStruct((n_tok, d_dict), jnp.float32),
        scratch_shapes=[
            pltpu.VMEM((2, nd, tb, dt), jnp.float32),
            pltpu.VMEM((2, tb, 1), jnp.float32),
            pltpu.VMEM((2, tb, 1), jnp.float32),
            pltpu.VMEM((tb, 1), jnp.float32),
            pltpu.VMEM((tb, 1), jnp.float32),
            pltpu.VMEM((tb, 1), jnp.float32),
        ],
    )(x, W_enc, b_enc2)

    x_hat = pl.pallas_call(
        functools.partial(_dec_kernel, nd=nd),
        grid=(nt, nd),
        in_specs=[
            pl.BlockSpec((tb, dt), lambda i, j: (i, j)),
            pl.BlockSpec((d_in, dt), lambda i, j: (0, j)),
            pl.BlockSpec((1, d_in), lambda i, j: (0, 0)),
        ],
        out_specs=pl.BlockSpec((tb, d_in), lambda i, j: (i, 0)),
        out_shape=jax.ShapeDtypeStruct((n_tok, d_in), jnp.float32),
        scratch_shapes=[pltpu.VMEM((tb, d_in), jnp.float32)],
    )(z_sparse, W_dec, b_dec2)

    return (x_hat, z_sparse)


# speedup vs baseline: 1.6321x; 1.2807x over previous
import functools

import jax
import jax.numpy as jnp
from jax.experimental import pallas as pl
from jax.experimental.pallas import tpu as pltpu

K = 32
F_ITERS = 2


def _count_ge(zv, mid):
    m = (zv >= mid[None, :, :]).astype(jnp.float32)
    c1 = jnp.sum(m, axis=2)
    return jnp.sum(c1, axis=0)[:, None]


def _fused_kernel(x_ref, we_ref, be_ref, wd_ref, bd_ref,
                  xh_ref, zsp_ref, z_s, thr_s, bm_lo, bm_hi,
                  st_lo, st_hi, st_cl, acc,
                  *, nt, nd, dt, tb):
    i = pl.program_id(0)
    j = pl.program_id(1)
    cur = jax.lax.rem(i, 2)
    prev = jax.lax.rem(i + 1, 2)
    kf = jnp.float32(K)
    qpt = -(-K // nd)
    cw = dt // qpt

    zold = z_s[cur, j]
    thr2 = thr_s[cur]
    zsp = jnp.where(zold >= thr2, zold, 0.0)
    zsp_ref[...] = zsp
    dec = jax.lax.dot_general(zsp, wd_ref[...], (((1,), (1,)), ((), ())),
                              preferred_element_type=jnp.float32)
    acc[...] = jnp.where(j == 0, dec, acc[...] + dec)

    @pl.when(j == nd - 1)
    def _emit_xhat():
        xh_ref[...] = acc[...] + bd_ref[...]

    z = jax.lax.dot_general(x_ref[...], we_ref[...], (((1,), (1,)), ((), ())),
                            preferred_element_type=jnp.float32)
    z = z + be_ref[...]
    z_s[cur, j] = z
    cmn = jnp.max(z[:, 0:cw], axis=1, keepdims=True)
    cmx = cmn
    for q in range(1, qpt):
        m_q = jnp.max(z[:, q * cw:(q + 1) * cw], axis=1, keepdims=True)
        cmn = jnp.minimum(cmn, m_q)
        cmx = jnp.maximum(cmx, m_q)
    bm_lo[cur] = jnp.where(j == 0, cmn, jnp.minimum(bm_lo[cur], cmn))
    bm_hi[cur] = jnp.where(j == 0, cmx, jnp.maximum(bm_hi[cur], cmx))

    zv = z_s[prev]
    lo = st_lo[...]
    hi = st_hi[...]
    cl = st_cl[...]
    for _ in range(F_ITERS):
        mid = 0.5 * (lo + hi)
        cnt = _count_ge(zv, mid)
        ge = cnt >= kf
        lo = jnp.where(ge, mid, lo)
        hi = jnp.where(ge, hi, mid)
        cl = jnp.where(ge, cnt, cl)
    st_lo[...] = lo
    st_hi[...] = hi
    st_cl[...] = cl

    @pl.when(jnp.logical_and(i >= 1, jnp.logical_and(i <= nt, j == nd - 1)))
    def _finish():
        def cond(c):
            it, lo_, hi_, cl_ = c
            return jnp.logical_and(it < 40, jnp.any(cl_ != kf))

        def body(c):
            it, lo_, hi_, cl_ = c
            mid = 0.5 * (lo_ + hi_)
            cnt = _count_ge(zv, mid)
            ge = cnt >= kf
            return (it + 1,
                    jnp.where(ge, mid, lo_),
                    jnp.where(ge, hi_, mid),
                    jnp.where(ge, cnt, cl_))

        _, thr, _, _ = jax.lax.while_loop(
            cond, body, (jnp.int32(0), st_lo[...], st_hi[...], st_cl[...]))
        thr_s[prev] = thr

    @pl.when(j == nd - 1)
    def _seed():
        st_lo[...] = bm_lo[cur]
        st_hi[...] = bm_hi[cur]
        st_cl[...] = jnp.full((tb, 1), kf + 1.0, jnp.float32)


def kernel(x, W_enc, b_enc, W_dec, b_dec):
    n_tok, d_in = x.shape
    d_dict = W_enc.shape[0]
    tb = min(256, n_tok)
    dt = 1024
    nt = n_tok // tb
    nd = d_dict // dt
    b_enc2 = b_enc.reshape(1, d_dict)
    b_dec2 = b_dec.reshape(1, d_in)

    x_hat, z_sparse = pl.pallas_call(
        functools.partial(_fused_kernel, nt=nt, nd=nd, dt=dt, tb=tb),
        grid=(nt + 2, nd),
        in_specs=[
            pl.BlockSpec((tb, d_in),
                         lambda i, j: (jnp.minimum(i, nt - 1), 0)),
            pl.BlockSpec((dt, d_in), lambda i, j: (j, 0)),
            pl.BlockSpec((1, dt), lambda i, j: (0, j)),
            pl.BlockSpec((d_in, dt), lambda i, j: (0, j)),
            pl.BlockSpec((1, d_in), lambda i, j: (0, 0)),
        ],
        out_specs=[
            pl.BlockSpec((tb, d_in), lambda i, j: (jnp.maximum(i - 2, 0), 0)),
            pl.BlockSpec((tb, dt),
                         lambda i, j: (jnp.maximum(i - 2, 0), j)),
        ],
        out_shape=[
            jax.ShapeDtypeStruct((n_tok, d_in), jnp.float32),
            jax.ShapeDtypeStruct((n_tok, d_dict), jnp.float32),
        ],
        scratch_shapes=[
            pltpu.VMEM((2, nd, tb, dt), jnp.float32),
            pltpu.VMEM((2, tb, 1), jnp.float32),
            pltpu.VMEM((2, tb, 1), jnp.float32),
            pltpu.VMEM((2, tb, 1), jnp.float32),
            pltpu.VMEM((tb, 1), jnp.float32),
            pltpu.VMEM((tb, 1), jnp.float32),
            pltpu.VMEM((tb, 1), jnp.float32),
            pltpu.VMEM((tb, d_in), jnp.float32),
        ],
    )(x, W_enc, b_enc2, W_dec, b_dec2)

    return (x_hat, z_sparse)
